# baseline (device time: 32128 ns/iter reference)
import jax
import jax.numpy as jnp
from jax import lax
from jax.experimental import pallas as pl
from jax.experimental.pallas import tpu as pltpu

N_DEV = 4


def kernel(x, router_W, route_idx, expert_W, shared_W):
    n_tok, d = x.shape
    n_exp = router_W.shape[1]
    e_per, _, h = expert_W.shape

    def body(x_ref, rw_ref, idx_ref, ew_ref, sw_ref, out_ref,
             own_ref, comm_ref, send_sems, recv_sems):
        my = lax.axis_index("i")
        left = lax.rem(my - 1 + N_DEV, N_DEV)
        right = lax.rem(my + 1, N_DEV)

        barrier_sem = pltpu.get_barrier_semaphore()
        for nbr in [left, right]:
            pl.semaphore_signal(
                barrier_sem, inc=1,
                device_id=(nbr,), device_id_type=pl.DeviceIdType.MESH,
            )
        pl.semaphore_wait(barrier_sem, 2)

        xf = x_ref[...]
        scores = jnp.dot(xf, rw_ref[...], preferred_element_type=jnp.float32)
        s_max = jnp.max(scores, axis=-1, keepdims=True)
        e = jnp.exp(scores - s_max)
        probs = e / jnp.sum(e, axis=-1, keepdims=True)

        idx = idx_ref[...]
        eids = lax.broadcasted_iota(jnp.int32, (n_tok, n_exp), 1)
        onehot = idx == eids
        p_tok = jnp.sum(jnp.where(onehot, probs, 0.0), axis=-1)

        xb = xf.astype(jnp.bfloat16)
        acc = jnp.zeros((n_tok, h), jnp.float32)
        for e_loc in range(e_per):
            ge = my * e_per + e_loc
            w = jnp.where(idx[:, 0] == ge, p_tok, 0.0)
            xw = xb * w.astype(jnp.bfloat16)[:, None]
            acc = acc + jnp.dot(
                xw, ew_ref[e_loc].astype(jnp.bfloat16),
                preferred_element_type=jnp.float32,
            )

        own_ref[...] = acc.astype(jnp.bfloat16)

        total = acc + jnp.dot(
            xb, sw_ref[...].astype(jnp.bfloat16),
            preferred_element_type=jnp.float32,
        )

        for t in range(N_DEV - 1):
            src = own_ref if t == 0 else comm_ref.at[t - 1]
            rdma = pltpu.make_async_remote_copy(
                src_ref=src,
                dst_ref=comm_ref.at[t],
                send_sem=send_sems.at[t],
                recv_sem=recv_sems.at[t],
                device_id=(right,),
                device_id_type=pl.DeviceIdType.MESH,
            )
            rdma.start()
            rdma.wait()
            total = total + comm_ref[t].astype(jnp.float32)

        out_ref[...] = total

    return pl.pallas_call(
        body,
        out_shape=jax.ShapeDtypeStruct((n_tok, h), jnp.float32),
        in_specs=[pl.BlockSpec(memory_space=pltpu.VMEM)] * 5,
        out_specs=pl.BlockSpec(memory_space=pltpu.VMEM),
        scratch_shapes=[
            pltpu.VMEM((n_tok, h), jnp.bfloat16),
            pltpu.VMEM((N_DEV - 1, n_tok, h), jnp.bfloat16),
            pltpu.SemaphoreType.DMA((N_DEV - 1,)),
            pltpu.SemaphoreType.DMA((N_DEV - 1,)),
        ],
        compiler_params=pltpu.CompilerParams(collective_id=0),
    )(x, router_W, route_idx, expert_W, shared_W)


# device time: 23775 ns/iter; 1.3513x vs baseline; 1.3513x over previous
import jax
import jax.numpy as jnp
from jax import lax
from jax.experimental import pallas as pl
from jax.experimental.pallas import tpu as pltpu

N_DEV = 4
N_HOP = N_DEV - 1


def kernel(x, router_W, route_idx, expert_W, shared_W):
    n_tok, d = x.shape
    n_exp = router_W.shape[1]
    e_per, _, h = expert_W.shape
    half = n_tok // 2

    def body(x_ref, rw_ref, idx_ref, ew_ref, sw_ref, out_ref,
             own_a, own_b, comm_a, comm_b,
             send_a, recv_a, send_b, recv_b):
        my = lax.axis_index("i")
        left = lax.rem(my - 1 + N_DEV, N_DEV)
        right = lax.rem(my + 1, N_DEV)

        barrier_sem = pltpu.get_barrier_semaphore()
        for nbr in [left, right]:
            pl.semaphore_signal(
                barrier_sem, inc=1,
                device_id=(nbr,), device_id_type=pl.DeviceIdType.MESH,
            )
        pl.semaphore_wait(barrier_sem, 2)

        xf = x_ref[...]
        scores = jnp.dot(xf, rw_ref[...], preferred_element_type=jnp.float32)
        s_max = jnp.max(scores, axis=-1, keepdims=True)
        e = jnp.exp(scores - s_max)
        probs = e / jnp.sum(e, axis=-1, keepdims=True)

        idx = idx_ref[...]
        eids = lax.broadcasted_iota(jnp.int32, (n_tok, n_exp), 1)
        onehot = idx == eids
        p_tok = jnp.sum(jnp.where(onehot, probs, 0.0), axis=-1)

        xb = xf.astype(jnp.bfloat16)
        acc = jnp.zeros((n_tok, h), jnp.float32)
        for e_loc in range(e_per):
            ge = my * e_per + e_loc
            w = jnp.where(idx[:, 0] == ge, p_tok, 0.0)
            xw = xb * w.astype(jnp.bfloat16)[:, None]
            acc = acc + jnp.dot(
                xw, ew_ref[e_loc].astype(jnp.bfloat16),
                preferred_element_type=jnp.float32,
            )

        accb = acc.astype(jnp.bfloat16)
        own_a[...] = accb[:half]
        own_b[...] = accb[half:]

        def hop(t, direction):
            own, comm, ssem, rsem, dst = (
                (own_a, comm_a, send_a, recv_a, right) if direction == 0
                else (own_b, comm_b, send_b, recv_b, left)
            )
            return pltpu.make_async_remote_copy(
                src_ref=own if t == 0 else comm.at[t - 1],
                dst_ref=comm.at[t],
                send_sem=ssem.at[t],
                recv_sem=rsem.at[t],
                device_id=(dst,),
                device_id_type=pl.DeviceIdType.MESH,
            )

        rdma = [[hop(0, 0), hop(0, 1)]]
        rdma[0][0].start()
        rdma[0][1].start()

        total = acc + jnp.dot(
            xb, sw_ref[...].astype(jnp.bfloat16),
            preferred_element_type=jnp.float32,
        )
        total_a, total_b = total[:half], total[half:]

        for t in range(N_HOP):
            rdma[t][0].wait_recv()
            rdma[t][1].wait_recv()
            if t + 1 < N_HOP:
                rdma.append([hop(t + 1, 0), hop(t + 1, 1)])
                rdma[t + 1][0].start()
                rdma[t + 1][1].start()
            total_a = total_a + comm_a[t].astype(jnp.float32)
            total_b = total_b + comm_b[t].astype(jnp.float32)
            rdma[t][0].wait_send()
            rdma[t][1].wait_send()

        out_ref[:half] = total_a
        out_ref[half:] = total_b

    return pl.pallas_call(
        body,
        out_shape=jax.ShapeDtypeStruct((n_tok, h), jnp.float32),
        in_specs=[pl.BlockSpec(memory_space=pltpu.VMEM)] * 5,
        out_specs=pl.BlockSpec(memory_space=pltpu.VMEM),
        scratch_shapes=[
            pltpu.VMEM((half, h), jnp.bfloat16),
            pltpu.VMEM((half, h), jnp.bfloat16),
            pltpu.VMEM((N_HOP, half, h), jnp.bfloat16),
            pltpu.VMEM((N_HOP, half, h), jnp.bfloat16),
            pltpu.SemaphoreType.DMA((N_HOP,)),
            pltpu.SemaphoreType.DMA((N_HOP,)),
            pltpu.SemaphoreType.DMA((N_HOP,)),
            pltpu.SemaphoreType.DMA((N_HOP,)),
        ],
        compiler_params=pltpu.CompilerParams(collective_id=0),
    )(x, router_W, route_idx, expert_W, shared_W)


# device time: 18890 ns/iter; 1.7008x vs baseline; 1.2586x over previous
import jax
import jax.numpy as jnp
from jax import lax
from jax.experimental import pallas as pl
from jax.experimental.pallas import tpu as pltpu

N_DEV = 4


def kernel(x, router_W, route_idx, expert_W, shared_W):
    n_tok, d = x.shape
    n_exp = router_W.shape[1]
    e_per, _, h = expert_W.shape
    half = n_tok // 2

    def body(x_ref, rw_ref, idx_ref, ew_ref, sw_ref, out_ref,
             snd_a1, snd_b1, snd_a2, snd_b2,
             rcv_a1, rcv_b1, rcv_a2, rcv_b2,
             send_sems, recv_sems):
        my = lax.axis_index("i")
        p1 = my ^ 1
        p2 = 3 - my

        barrier_sem = pltpu.get_barrier_semaphore()
        for nbr in [p1, p2]:
            pl.semaphore_signal(
                barrier_sem, inc=1,
                device_id=(nbr,), device_id_type=pl.DeviceIdType.MESH,
            )
        pl.semaphore_wait(barrier_sem, 2)

        def xchg(src, dst, sem_ix, partner):
            r = pltpu.make_async_remote_copy(
                src_ref=src, dst_ref=dst,
                send_sem=send_sems.at[sem_ix], recv_sem=recv_sems.at[sem_ix],
                device_id=(partner,), device_id_type=pl.DeviceIdType.MESH,
            )
            r.start()
            return r

        xf = x_ref[...]
        scores = jnp.dot(xf, rw_ref[...], preferred_element_type=jnp.float32)
        s_max = jnp.max(scores, axis=-1, keepdims=True)
        e = jnp.exp(scores - s_max)
        probs = e / jnp.sum(e, axis=-1, keepdims=True)

        idx = idx_ref[...]
        eids = lax.broadcasted_iota(jnp.int32, (n_tok, n_exp), 1)
        onehot = idx == eids
        p_tok = jnp.sum(jnp.where(onehot, probs, 0.0), axis=-1)

        xb = xf.astype(jnp.bfloat16)
        acc = jnp.zeros((n_tok, h), jnp.float32)
        for e_loc in range(e_per):
            ge = my * e_per + e_loc
            w = jnp.where(idx[:, 0] == ge, p_tok, 0.0)
            xw = xb * w.astype(jnp.bfloat16)[:, None]
            acc = acc + jnp.dot(
                xw, ew_ref[e_loc].astype(jnp.bfloat16),
                preferred_element_type=jnp.float32,
            )

        accb = acc.astype(jnp.bfloat16)
        snd_a1[...] = accb[:half]
        snd_b1[...] = accb[half:]

        a1 = xchg(snd_a1, rcv_a1, 0, p2)
        b1 = xchg(snd_b1, rcv_b1, 1, p1)

        shared = jnp.dot(
            xb, sw_ref[...].astype(jnp.bfloat16),
            preferred_element_type=jnp.float32,
        )

        a1.wait_recv()
        snd_a2[...] = accb[:half] + rcv_a1[...]
        a2 = xchg(snd_a2, rcv_a2, 2, p1)
        b1.wait_recv()
        snd_b2[...] = accb[half:] + rcv_b1[...]
        b2 = xchg(snd_b2, rcv_b2, 3, p2)

        a2.wait_recv()
        out_ref[:half] = (
            shared[:half]
            + (snd_a2[...] + rcv_a2[...]).astype(jnp.float32)
        )
        b2.wait_recv()
        out_ref[half:] = (
            shared[half:]
            + (snd_b2[...] + rcv_b2[...]).astype(jnp.float32)
        )

        for r in (a1, b1, a2, b2):
            r.wait_send()

    return pl.pallas_call(
        body,
        out_shape=jax.ShapeDtypeStruct((n_tok, h), jnp.float32),
        in_specs=[pl.BlockSpec(memory_space=pltpu.VMEM)] * 5,
        out_specs=pl.BlockSpec(memory_space=pltpu.VMEM),
        scratch_shapes=(
            [pltpu.VMEM((half, h), jnp.bfloat16)] * 8
            + [pltpu.SemaphoreType.DMA((4,))] * 2
        ),
        compiler_params=pltpu.CompilerParams(collective_id=0),
    )(x, router_W, route_idx, expert_W, shared_W)


# device time: 18661 ns/iter; 1.7217x vs baseline; 1.0123x over previous
import jax
import jax.numpy as jnp
from jax import lax
from jax.experimental import pallas as pl
from jax.experimental.pallas import tpu as pltpu

N_DEV = 4


def kernel(x, router_W, route_idx, expert_W, shared_W):
    n_tok, d = x.shape
    n_exp = router_W.shape[1]
    e_per, _, h = expert_W.shape
    half = n_tok // 2

    def body(x_ref, rw_ref, idx_ref, ew_ref, sw_ref, out_ref,
             snd_a1, snd_b1, snd_a2, snd_b2,
             rcv_a1, rcv_b1, rcv_a2, rcv_b2,
             send_sems, recv_sems):
        my = lax.axis_index("i")
        p1 = my ^ 1
        p2 = 3 - my

        barrier_sem = pltpu.get_barrier_semaphore()
        for nbr in [p1, p2]:
            pl.semaphore_signal(
                barrier_sem, inc=1,
                device_id=(nbr,), device_id_type=pl.DeviceIdType.MESH,
            )
        pl.semaphore_wait(barrier_sem, 2)

        def xchg(src, dst, sem_ix, partner):
            r = pltpu.make_async_remote_copy(
                src_ref=src, dst_ref=dst,
                send_sem=send_sems.at[sem_ix], recv_sem=recv_sems.at[sem_ix],
                device_id=(partner,), device_id_type=pl.DeviceIdType.MESH,
            )
            r.start()
            return r

        xf = x_ref[...]
        scores = jnp.dot(xf, rw_ref[...], preferred_element_type=jnp.float32)
        s_max = jnp.max(scores, axis=-1, keepdims=True)
        e = jnp.exp(scores - s_max)
        probs = e / jnp.sum(e, axis=-1, keepdims=True)

        idx = idx_ref[...]
        eids = lax.broadcasted_iota(jnp.int32, (n_tok, n_exp), 1)
        onehot = idx == eids
        p_tok = jnp.sum(jnp.where(onehot, probs, 0.0), axis=-1)

        xb = xf.astype(jnp.bfloat16)
        ewb = [ew_ref[e_loc].astype(jnp.bfloat16) for e_loc in range(e_per)]
        xw = []
        for e_loc in range(e_per):
            ge = my * e_per + e_loc
            w = jnp.where(idx[:, 0] == ge, p_tok, 0.0)
            xw.append(xb * w.astype(jnp.bfloat16)[:, None])

        def partial_rows(lo, hi):
            acc = jnp.zeros((hi - lo, h), jnp.float32)
            for e_loc in range(e_per):
                acc = acc + jnp.dot(
                    xw[e_loc][lo:hi], ewb[e_loc],
                    preferred_element_type=jnp.float32,
                )
            return acc.astype(jnp.bfloat16)

        acc_a = partial_rows(0, half)
        snd_a1[...] = acc_a
        a1 = xchg(snd_a1, rcv_a1, 0, p2)
        acc_b = partial_rows(half, n_tok)
        snd_b1[...] = acc_b
        b1 = xchg(snd_b1, rcv_b1, 1, p1)

        shared = jnp.dot(
            xb, sw_ref[...].astype(jnp.bfloat16),
            preferred_element_type=jnp.float32,
        )

        a1.wait_recv()
        snd_a2[...] = acc_a + rcv_a1[...]
        a2 = xchg(snd_a2, rcv_a2, 2, p1)
        b1.wait_recv()
        snd_b2[...] = acc_b + rcv_b1[...]
        b2 = xchg(snd_b2, rcv_b2, 3, p2)

        a2.wait_recv()
        out_ref[:half] = (
            shared[:half]
            + (snd_a2[...] + rcv_a2[...]).astype(jnp.float32)
        )
        b2.wait_recv()
        out_ref[half:] = (
            shared[half:]
            + (snd_b2[...] + rcv_b2[...]).astype(jnp.float32)
        )

        for r in (a1, b1, a2, b2):
            r.wait_send()

    return pl.pallas_call(
        body,
        out_shape=jax.ShapeDtypeStruct((n_tok, h), jnp.float32),
        in_specs=[pl.BlockSpec(memory_space=pltpu.VMEM)] * 5,
        out_specs=pl.BlockSpec(memory_space=pltpu.VMEM),
        scratch_shapes=(
            [pltpu.VMEM((half, h), jnp.bfloat16)] * 8
            + [pltpu.SemaphoreType.DMA((4,))] * 2
        ),
        compiler_params=pltpu.CompilerParams(collective_id=0),
    )(x, router_W, route_idx, expert_W, shared_W)
